# baseline (speedup 1.0000x reference)
import functools

import jax
import jax.numpy as jnp
from jax import lax
from jax.experimental import pallas as pl
from jax.experimental.pallas import tpu as pltpu
from jax.experimental.pallas import tpu_sc as plsc

VOCAB = 1000000
DIM = 64
B = 1024
L = 50
NC = 20

NSETS = NC + 2
NENC = NSETS * B
NWORKERS = 32
ENC_PER_W = NENC // NWORKERS
CHUNK = 16
NSTEPS = ENC_PER_W // CHUNK
ROWS = CHUNK * L
GW = 80
NGATHER = ROWS // GW
LANES = 16
DSUB = DIM // LANES


def _sc_encode_sums(idx, table, nenc, enc_off):
    epw = nenc // NWORKERS
    nsteps = epw // CHUNK
    mesh = plsc.VectorSubcoreMesh(core_axis_name="c", subcore_axis_name="s")

    @functools.partial(
        pl.kernel,
        out_type=jax.ShapeDtypeStruct((nenc, DIM), jnp.float32),
        mesh=mesh,
        scratch_types=[
            pltpu.VMEM((2, ROWS), jnp.int32),
            pltpu.VMEM((2, ROWS, DIM), jnp.float32),
            pltpu.VMEM((2, CHUNK, DIM), jnp.float32),
            pltpu.SemaphoreType.DMA,
            pltpu.SemaphoreType.DMA,
            pltpu.SemaphoreType.DMA,
            pltpu.SemaphoreType.DMA,
            pltpu.SemaphoreType.DMA,
            pltpu.SemaphoreType.DMA,
        ],
        compiler_params=pltpu.CompilerParams(use_tc_tiling_on_sc=False),
    )
    def sc_kernel(idx_hbm, table_hbm, out_hbm, idx_v, rows_v, out_v,
                  gsem0, gsem1, isem0, isem1, osem0, osem1):
        wid = lax.axis_index("s") * 2 + lax.axis_index("c")
        gsems = (gsem0, gsem1)
        isems = (isem0, isem1)
        osems = (osem0, osem1)

        def idx_copy(s, b):
            return pltpu.make_async_copy(
                idx_hbm.at[pl.ds((enc_off + wid * epw + s * CHUNK) * L, ROWS)],
                idx_v.at[b], isems[b])

        def out_copy(s, b):
            return pltpu.make_async_copy(
                out_v.at[b],
                out_hbm.at[pl.ds(wid * epw + s * CHUNK, CHUNK)],
                osems[b])

        def gather_copy(c, b):
            return pltpu.make_async_copy(
                table_hbm.at[idx_v.at[b, pl.ds(c * GW, GW)]],
                rows_v.at[b, pl.ds(c * GW, GW)], gsems[b])

        def fire(b):
            for c in range(NGATHER):
                gather_copy(c, b).start()

        def drain(b):
            for c in range(NGATHER):
                gather_copy(c, b).wait()

        def reduce(s, b):
            @pl.when(s >= 2)
            def _():
                out_copy(s, b).wait()

            @pl.loop(0, CHUNK)
            def _enc(e):
                base = e * L
                for c4 in range(DSUB):
                    acc = rows_v[b, base, pl.ds(c4 * LANES, LANES)]
                    for l in range(1, L):
                        acc = acc + rows_v[b, base + l, pl.ds(c4 * LANES, LANES)]
                    out_v[b, e, pl.ds(c4 * LANES, LANES)] = acc

            out_copy(s, b).start()

        idx_copy(0, 0).start()
        idx_copy(1, 1).start()
        idx_copy(0, 0).wait()
        fire(0)

        @pl.loop(0, nsteps, step=2)
        def _step(s):
            idx_copy(s + 1, 1).wait()
            fire(1)
            drain(0)

            @pl.when(s + 2 < nsteps)
            def _():
                idx_copy(s + 2, 0).start()

            reduce(s, 0)

            @pl.when(s + 2 < nsteps)
            def _():
                idx_copy(s + 2, 0).wait()
                fire(0)

            drain(1)

            @pl.when(s + 3 < nsteps)
            def _():
                idx_copy(s + 3, 1).start()

            reduce(s + 1, 1)

        out_copy(nsteps - 2, 0).wait()
        out_copy(nsteps - 1, 1).wait()

    return sc_kernel(idx, table)


RELAY_CB = 8192
RELAY_GRID = -(-VOCAB // RELAY_CB)
V_PAD = RELAY_GRID * RELAY_CB
HB = RELAY_CB // 2


def _tc_relayout(tt):
    def body(t_ref, o_ref):
        y = jnp.transpose(t_ref[...])
        o_ref[:, 0:DIM] = y[0:HB]
        o_ref[:, DIM:2 * DIM] = y[HB:2 * HB]

    return pl.pallas_call(
        body,
        grid=(RELAY_GRID,),
        in_specs=[pl.BlockSpec((DIM, RELAY_CB), lambda i: (0, i))],
        out_specs=pl.BlockSpec((HB, 2 * DIM), lambda i: (i, 0)),
        out_shape=jax.ShapeDtypeStruct((V_PAD // 2, 2 * DIM), jnp.float32),
    )(tt)


def _remap_idx(r):
    return ((r >> 13) << 13) | ((r & 4095) << 1) | ((r >> 12) & 1)


NBLK = NSETS - 1
NSETS_A = 18
NENC_A = NSETS_A * B
NENC_B = NENC - NENC_A
NBLK_A = NSETS_A - 1


def _score_body(x_ref, e_ref, o_ref):
    inv = 1.0 / float(L * L)
    s = lax.dot_general(
        x_ref[...] * inv, e_ref[...], (((1,), (1,)), ((), ())),
        preferred_element_type=jnp.float32,
    )
    m = jnp.max(s, axis=1, keepdims=True)
    p = jnp.exp(s - m)
    o_ref[...] = p / jnp.sum(p, axis=1, keepdims=True)


def _tc_score_a(sums_a):
    return pl.pallas_call(
        _score_body,
        grid=(NBLK_A,),
        in_specs=[
            pl.BlockSpec((B, DIM), lambda k: (0, 0)),
            pl.BlockSpec((B, DIM), lambda k: (k + 1, 0)),
        ],
        out_specs=pl.BlockSpec((B, B), lambda k: (k, 0)),
        out_shape=jax.ShapeDtypeStruct((NBLK * B, B), jnp.float32),
    )(sums_a, sums_a)


def _tc_score_b(sums_a, sums_b, prev):
    def body(x_ref, e_ref, prev_ref, o_ref):
        del prev_ref
        _score_body(x_ref, e_ref, o_ref)

    return pl.pallas_call(
        body,
        grid=(NBLK - NBLK_A,),
        in_specs=[
            pl.BlockSpec((B, DIM), lambda j: (0, 0)),
            pl.BlockSpec((B, DIM), lambda j: (j, 0)),
            pl.BlockSpec(memory_space=pl.ANY),
        ],
        out_specs=pl.BlockSpec((B, B), lambda j: (NBLK_A + j, 0)),
        out_shape=jax.ShapeDtypeStruct((NBLK * B, B), jnp.float32),
        input_output_aliases={2: 0},
    )(sums_a, sums_b, prev)


def kernel(xs, ys, cands, table):
    idx = jnp.concatenate(
        [xs.reshape(-1), ys.reshape(-1), cands.reshape(-1)]
    ).astype(jnp.int32)
    idx = _remap_idx(idx)
    t_lin = _tc_relayout(table.T).reshape(V_PAD, DIM)
    sums_a = _sc_encode_sums(idx, t_lin, NENC_A, 0)
    sums_b = _sc_encode_sums(idx, t_lin, NENC_B, NENC_A)
    part = _tc_score_a(sums_a)
    pred = _tc_score_b(sums_a, sums_b, part)
    return pred[None]


# speedup vs baseline: 1.0302x; 1.0302x over previous
import functools

import jax
import jax.numpy as jnp
from jax import lax
from jax.experimental import pallas as pl
from jax.experimental.pallas import tpu as pltpu
from jax.experimental.pallas import tpu_sc as plsc

VOCAB = 1000000
DIM = 64
B = 1024
L = 50
NC = 20

NSETS = NC + 2
NENC = NSETS * B
NWORKERS = 32
ENC_PER_W = NENC // NWORKERS
CHUNK = 16
NSTEPS = ENC_PER_W // CHUNK
ROWS = CHUNK * L
GW = 80
NGATHER = ROWS // GW
LANES = 16
DSUB = DIM // LANES


def _sc_encode_sums(idx, table, nenc, enc_off):
    epw = nenc // NWORKERS
    nsteps = epw // CHUNK
    mesh = plsc.VectorSubcoreMesh(core_axis_name="c", subcore_axis_name="s")

    @functools.partial(
        pl.kernel,
        out_type=jax.ShapeDtypeStruct((nenc, DIM), jnp.float32),
        mesh=mesh,
        scratch_types=[
            pltpu.VMEM((2, ROWS), jnp.int32),
            pltpu.VMEM((2, ROWS, DIM), jnp.float32),
            pltpu.VMEM((2, CHUNK, DIM), jnp.float32),
            pltpu.SemaphoreType.DMA,
            pltpu.SemaphoreType.DMA,
            pltpu.SemaphoreType.DMA,
            pltpu.SemaphoreType.DMA,
            pltpu.SemaphoreType.DMA,
            pltpu.SemaphoreType.DMA,
        ],
        compiler_params=pltpu.CompilerParams(use_tc_tiling_on_sc=False),
    )
    def sc_kernel(idx_hbm, table_hbm, out_hbm, idx_v, rows_v, out_v,
                  gsem0, gsem1, isem0, isem1, osem0, osem1):
        wid = lax.axis_index("s") * 2 + lax.axis_index("c")
        gsems = (gsem0, gsem1)
        isems = (isem0, isem1)
        osems = (osem0, osem1)

        def idx_copy(s, b):
            return pltpu.make_async_copy(
                idx_hbm.at[pl.ds((enc_off + wid * epw + s * CHUNK) * L, ROWS)],
                idx_v.at[b], isems[b])

        def out_copy(s, b):
            return pltpu.make_async_copy(
                out_v.at[b],
                out_hbm.at[pl.ds(wid * epw + s * CHUNK, CHUNK)],
                osems[b])

        def gather_copy(c, b):
            return pltpu.make_async_copy(
                table_hbm.at[idx_v.at[b, pl.ds(c * GW, GW)]],
                rows_v.at[b, pl.ds(c * GW, GW)], gsems[b])

        def fire(b):
            for c in range(NGATHER):
                gather_copy(c, b).start()

        def drain(b):
            for c in range(NGATHER):
                gather_copy(c, b).wait()

        def reduce(s, b):
            @pl.when(s >= 2)
            def _():
                out_copy(s, b).wait()

            @pl.loop(0, CHUNK)
            def _enc(e):
                base = e * L
                for c4 in range(DSUB):
                    acc = rows_v[b, base, pl.ds(c4 * LANES, LANES)]
                    for l in range(1, L):
                        acc = acc + rows_v[b, base + l, pl.ds(c4 * LANES, LANES)]
                    out_v[b, e, pl.ds(c4 * LANES, LANES)] = acc

            out_copy(s, b).start()

        idx_copy(0, 0).start()
        idx_copy(1, 1).start()
        idx_copy(0, 0).wait()
        fire(0)

        @pl.loop(0, nsteps, step=2)
        def _step(s):
            idx_copy(s + 1, 1).wait()
            fire(1)
            drain(0)

            @pl.when(s + 2 < nsteps)
            def _():
                idx_copy(s + 2, 0).start()

            reduce(s, 0)

            @pl.when(s + 2 < nsteps)
            def _():
                idx_copy(s + 2, 0).wait()
                fire(0)

            drain(1)

            @pl.when(s + 3 < nsteps)
            def _():
                idx_copy(s + 3, 1).start()

            reduce(s + 1, 1)

        out_copy(nsteps - 2, 0).wait()
        out_copy(nsteps - 1, 1).wait()

    return sc_kernel(idx, table)


RELAY_CB = 8192
RELAY_GRID = -(-VOCAB // RELAY_CB)
V_PAD = RELAY_GRID * RELAY_CB
HB = RELAY_CB // 2


def _tc_relayout(tt):
    def body(t_ref, o_ref):
        y = jnp.transpose(t_ref[...])
        o_ref[:, 0:DIM] = y[0:HB]
        o_ref[:, DIM:2 * DIM] = y[HB:2 * HB]

    return pl.pallas_call(
        body,
        grid=(RELAY_GRID,),
        in_specs=[pl.BlockSpec((DIM, RELAY_CB), lambda i: (0, i))],
        out_specs=pl.BlockSpec((HB, 2 * DIM), lambda i: (i, 0)),
        out_shape=jax.ShapeDtypeStruct((V_PAD // 2, 2 * DIM), jnp.float32),
    )(tt)


def _remap_idx(r):
    return ((r >> 13) << 13) | ((r & 4095) << 1) | ((r >> 12) & 1)


NBLK = NSETS - 1


def _tc_score_softmax(sums):
    inv = 1.0 / float(L * L)

    def body(x_ref, e_ref, o_ref):
        s = lax.dot_general(
            x_ref[...] * inv, e_ref[...], (((1,), (1,)), ((), ())),
            preferred_element_type=jnp.float32,
        )
        m = jnp.max(s, axis=1, keepdims=True)
        p = jnp.exp(s - m)
        o_ref[...] = p / jnp.sum(p, axis=1, keepdims=True)

    return pl.pallas_call(
        body,
        grid=(NBLK,),
        in_specs=[
            pl.BlockSpec((B, DIM), lambda k: (0, 0)),
            pl.BlockSpec((B, DIM), lambda k: (k + 1, 0)),
        ],
        out_specs=pl.BlockSpec((B, B), lambda k: (k, 0)),
        out_shape=jax.ShapeDtypeStruct((NBLK * B, B), jnp.float32),
    )(sums, sums)


def kernel(xs, ys, cands, table):
    idx = jnp.concatenate(
        [xs.reshape(-1), ys.reshape(-1), cands.reshape(-1)]
    ).astype(jnp.int32)
    idx = _remap_idx(idx)
    t_lin = _tc_relayout(table.T).reshape(V_PAD, DIM)
    sums = _sc_encode_sums(idx, t_lin, NENC, 0)
    pred = _tc_score_softmax(sums)
    return pred[None]


# speedup vs baseline: 1.0937x; 1.0616x over previous
import functools

import jax
import jax.numpy as jnp
from jax import lax
from jax.experimental import pallas as pl
from jax.experimental.pallas import tpu as pltpu
from jax.experimental.pallas import tpu_sc as plsc

VOCAB = 1000000
DIM = 64
B = 1024
L = 50
NC = 20

NSETS = NC + 2
NENC = NSETS * B
NWORKERS = 32
ENC_PER_W = NENC // NWORKERS
CHUNK = 16
NSTEPS = ENC_PER_W // CHUNK
ROWS = CHUNK * L
GW = 80
NGATHER = ROWS // GW
LANES = 16
DSUB = DIM // LANES


def _sc_encode_sums(idx, table, nenc, enc_off):
    epw = nenc // NWORKERS
    nsteps = epw // CHUNK
    mesh = plsc.VectorSubcoreMesh(core_axis_name="c", subcore_axis_name="s")

    @functools.partial(
        pl.kernel,
        out_type=jax.ShapeDtypeStruct((nenc, DIM), jnp.float32),
        mesh=mesh,
        scratch_types=[
            pltpu.VMEM((2, ROWS), jnp.int32),
            pltpu.VMEM((2, ROWS, DIM), jnp.float32),
            pltpu.VMEM((2, CHUNK, DIM), jnp.float32),
            pltpu.SemaphoreType.DMA,
            pltpu.SemaphoreType.DMA,
            pltpu.SemaphoreType.DMA,
            pltpu.SemaphoreType.DMA,
            pltpu.SemaphoreType.DMA,
            pltpu.SemaphoreType.DMA,
        ],
        compiler_params=pltpu.CompilerParams(use_tc_tiling_on_sc=False),
    )
    def sc_kernel(idx_hbm, table_hbm, out_hbm, idx_v, rows_v, out_v,
                  gsem0, gsem1, isem0, isem1, osem0, osem1):
        wid = lax.axis_index("s") * 2 + lax.axis_index("c")
        gsems = (gsem0, gsem1)
        isems = (isem0, isem1)
        osems = (osem0, osem1)

        def idx_copy(s, b):
            return pltpu.make_async_copy(
                idx_hbm.at[pl.ds((enc_off + wid * epw + s * CHUNK) * L, ROWS)],
                idx_v.at[b], isems[b])

        def out_copy(s, b):
            return pltpu.make_async_copy(
                out_v.at[b],
                out_hbm.at[pl.ds(wid * epw + s * CHUNK, CHUNK)],
                osems[b])

        def gather_copy(c, b):
            return pltpu.make_async_copy(
                table_hbm.at[idx_v.at[b, pl.ds(c * GW, GW)]],
                rows_v.at[b, pl.ds(c * GW, GW)], gsems[b])

        def fire(b):
            for c in range(NGATHER):
                gather_copy(c, b).start()

        def drain(b):
            for c in range(NGATHER):
                gather_copy(c, b).wait()

        def reduce(s, b):
            @pl.when(s >= 2)
            def _():
                out_copy(s, b).wait()

            @pl.loop(0, CHUNK)
            def _enc(e):
                base = e * L
                for c4 in range(DSUB):
                    acc = rows_v[b, base, pl.ds(c4 * LANES, LANES)]
                    for l in range(1, L):
                        acc = acc + rows_v[b, base + l, pl.ds(c4 * LANES, LANES)]
                    out_v[b, e, pl.ds(c4 * LANES, LANES)] = acc

            out_copy(s, b).start()

        idx_copy(0, 0).start()
        idx_copy(1, 1).start()
        idx_copy(0, 0).wait()
        fire(0)

        @pl.loop(0, nsteps, step=2)
        def _step(s):
            idx_copy(s + 1, 1).wait()
            fire(1)
            drain(0)

            @pl.when(s + 2 < nsteps)
            def _():
                idx_copy(s + 2, 0).start()

            reduce(s, 0)

            @pl.when(s + 2 < nsteps)
            def _():
                idx_copy(s + 2, 0).wait()
                fire(0)

            drain(1)

            @pl.when(s + 3 < nsteps)
            def _():
                idx_copy(s + 3, 1).start()

            reduce(s + 1, 1)

        out_copy(nsteps - 2, 0).wait()
        out_copy(nsteps - 1, 1).wait()

    return sc_kernel(idx, table)


RELAY_CB = 16384
RELAY_GRID = -(-VOCAB // RELAY_CB)
V_PAD = RELAY_GRID * RELAY_CB
HB = RELAY_CB // 2


def _tc_relayout(tt):
    def body(t_ref, o_ref):
        y = jnp.transpose(t_ref[...])
        o_ref[:, 0:DIM] = y[0:HB]
        o_ref[:, DIM:2 * DIM] = y[HB:2 * HB]

    return pl.pallas_call(
        body,
        grid=(RELAY_GRID,),
        in_specs=[pl.BlockSpec((DIM, RELAY_CB), lambda i: (0, i))],
        out_specs=pl.BlockSpec((HB, 2 * DIM), lambda i: (i, 0)),
        out_shape=jax.ShapeDtypeStruct((V_PAD // 2, 2 * DIM), jnp.float32),
    )(tt)


RELAY_SH = RELAY_CB.bit_length() - 1


def _remap_idx(r):
    return ((r >> RELAY_SH) << RELAY_SH) | ((r & (HB - 1)) << 1) | (
        (r >> (RELAY_SH - 1)) & 1)


NBLK = NSETS - 1


def _tc_score_softmax(sums):
    inv = 1.0 / float(L * L)

    def body(x_ref, e_ref, o_ref):
        s = lax.dot_general(
            x_ref[...] * inv, e_ref[...], (((1,), (1,)), ((), ())),
            preferred_element_type=jnp.float32,
        )
        m = jnp.max(s, axis=1, keepdims=True)
        p = jnp.exp(s - m)
        o_ref[...] = p / jnp.sum(p, axis=1, keepdims=True)

    return pl.pallas_call(
        body,
        grid=(NBLK,),
        in_specs=[
            pl.BlockSpec((B, DIM), lambda k: (0, 0)),
            pl.BlockSpec((B, DIM), lambda k: (k + 1, 0)),
        ],
        out_specs=pl.BlockSpec((B, B), lambda k: (k, 0)),
        out_shape=jax.ShapeDtypeStruct((NBLK * B, B), jnp.float32),
    )(sums, sums)


def kernel(xs, ys, cands, table):
    idx = jnp.concatenate(
        [xs.reshape(-1), ys.reshape(-1), cands.reshape(-1)]
    ).astype(jnp.int32)
    idx = _remap_idx(idx)
    t_lin = _tc_relayout(table.T).reshape(V_PAD, DIM)
    sums = _sc_encode_sums(idx, t_lin, NENC, 0)
    pred = _tc_score_softmax(sums)
    return pred[None]


# speedup vs baseline: 1.1260x; 1.0295x over previous
import functools

import jax
import jax.numpy as jnp
from jax import lax
from jax.experimental import pallas as pl
from jax.experimental.pallas import tpu as pltpu
from jax.experimental.pallas import tpu_sc as plsc

VOCAB = 1000000
DIM = 64
B = 1024
L = 50
NC = 20

NSETS = NC + 2
NENC = NSETS * B
NWORKERS = 32
ENC_PER_W = NENC // NWORKERS
CHUNK = 16
NSTEPS = ENC_PER_W // CHUNK
ROWS = CHUNK * L
GW = 80
NGATHER = ROWS // GW
LANES = 16
DSUB = DIM // LANES


def _sc_encode_sums(idx, table, nenc, enc_off):
    epw = nenc // NWORKERS
    nsteps = epw // CHUNK
    mesh = plsc.VectorSubcoreMesh(core_axis_name="c", subcore_axis_name="s")

    @functools.partial(
        pl.kernel,
        out_type=jax.ShapeDtypeStruct((nenc, DIM), jnp.float32),
        mesh=mesh,
        scratch_types=[
            pltpu.VMEM((2, ROWS), jnp.int32),
            pltpu.VMEM((2, ROWS, DIM), jnp.float32),
            pltpu.VMEM((2, CHUNK, DIM), jnp.float32),
            pltpu.SemaphoreType.DMA,
            pltpu.SemaphoreType.DMA,
            pltpu.SemaphoreType.DMA,
            pltpu.SemaphoreType.DMA,
            pltpu.SemaphoreType.DMA,
            pltpu.SemaphoreType.DMA,
        ],
        compiler_params=pltpu.CompilerParams(use_tc_tiling_on_sc=False),
    )
    def sc_kernel(idx_hbm, table_hbm, out_hbm, idx_v, rows_v, out_v,
                  gsem0, gsem1, isem0, isem1, osem0, osem1):
        wid = lax.axis_index("s") * 2 + lax.axis_index("c")
        gsems = (gsem0, gsem1)
        isems = (isem0, isem1)
        osems = (osem0, osem1)

        def idx_copy(s, b):
            return pltpu.make_async_copy(
                idx_hbm.at[pl.ds((enc_off + wid * epw + s * CHUNK) * L, ROWS)],
                idx_v.at[b], isems[b])

        def out_copy(s, b):
            return pltpu.make_async_copy(
                out_v.at[b],
                out_hbm.at[pl.ds(wid * epw + s * CHUNK, CHUNK)],
                osems[b])

        def gather_copy(c, b):
            return pltpu.make_async_copy(
                table_hbm.at[idx_v.at[b, pl.ds(c * GW, GW)]],
                rows_v.at[b, pl.ds(c * GW, GW)], gsems[b])

        def fire(b):
            for c in range(NGATHER):
                gather_copy(c, b).start()

        def drain(b):
            for c in range(NGATHER):
                gather_copy(c, b).wait()

        def reduce(s, b):
            @pl.when(s >= 2)
            def _():
                out_copy(s, b).wait()

            @pl.loop(0, CHUNK)
            def _enc(e):
                base = e * L
                for c4 in range(DSUB):
                    acc = rows_v[b, base, pl.ds(c4 * LANES, LANES)]
                    for l in range(1, L):
                        acc = acc + rows_v[b, base + l, pl.ds(c4 * LANES, LANES)]
                    out_v[b, e, pl.ds(c4 * LANES, LANES)] = acc

            out_copy(s, b).start()

        idx_copy(0, 0).start()
        idx_copy(1, 1).start()
        idx_copy(0, 0).wait()
        fire(0)

        @pl.loop(0, nsteps, step=2)
        def _step(s):
            idx_copy(s + 1, 1).wait()
            fire(1)
            drain(0)

            @pl.when(s + 2 < nsteps)
            def _():
                idx_copy(s + 2, 0).start()

            reduce(s, 0)

            @pl.when(s + 2 < nsteps)
            def _():
                idx_copy(s + 2, 0).wait()
                fire(0)

            drain(1)

            @pl.when(s + 3 < nsteps)
            def _():
                idx_copy(s + 3, 1).start()

            reduce(s + 1, 1)

        out_copy(nsteps - 2, 0).wait()
        out_copy(nsteps - 1, 1).wait()

    return sc_kernel(idx, table)


RELAY_CB = 32768
RELAY_GRID = -(-VOCAB // RELAY_CB)
V_PAD = RELAY_GRID * RELAY_CB
HB = RELAY_CB // 2


def _tc_relayout(tt):
    def body(t_ref, o_ref):
        y = jnp.transpose(t_ref[...])
        o_ref[:, 0:DIM] = y[0:HB]
        o_ref[:, DIM:2 * DIM] = y[HB:2 * HB]

    return pl.pallas_call(
        body,
        grid=(RELAY_GRID,),
        in_specs=[pl.BlockSpec((DIM, RELAY_CB), lambda i: (0, i))],
        out_specs=pl.BlockSpec((HB, 2 * DIM), lambda i: (i, 0)),
        out_shape=jax.ShapeDtypeStruct((V_PAD // 2, 2 * DIM), jnp.float32),
    )(tt)


RELAY_SH = RELAY_CB.bit_length() - 1


def _remap_idx(r):
    return ((r >> RELAY_SH) << RELAY_SH) | ((r & (HB - 1)) << 1) | (
        (r >> (RELAY_SH - 1)) & 1)


NBLK = NSETS - 1


def _tc_score_softmax(sums):
    inv = 1.0 / float(L * L)

    def body(x_ref, e_ref, o_ref):
        s = lax.dot_general(
            x_ref[...] * inv, e_ref[...], (((1,), (1,)), ((), ())),
            preferred_element_type=jnp.float32,
        )
        m = jnp.max(s, axis=1, keepdims=True)
        p = jnp.exp(s - m)
        o_ref[...] = p / jnp.sum(p, axis=1, keepdims=True)

    return pl.pallas_call(
        body,
        grid=(NBLK,),
        in_specs=[
            pl.BlockSpec((B, DIM), lambda k: (0, 0)),
            pl.BlockSpec((B, DIM), lambda k: (k + 1, 0)),
        ],
        out_specs=pl.BlockSpec((B, B), lambda k: (k, 0)),
        out_shape=jax.ShapeDtypeStruct((NBLK * B, B), jnp.float32),
    )(sums, sums)


def kernel(xs, ys, cands, table):
    idx = jnp.concatenate(
        [xs.reshape(-1), ys.reshape(-1), cands.reshape(-1)]
    ).astype(jnp.int32)
    idx = _remap_idx(idx)
    t_lin = _tc_relayout(table.T).reshape(V_PAD, DIM)
    sums = _sc_encode_sums(idx, t_lin, NENC, 0)
    pred = _tc_score_softmax(sums)
    return pred[None]
